# baseline (speedup 1.0000x reference)
import functools

import jax
import jax.numpy as jnp
from jax import lax
from jax.experimental import pallas as pl
from jax.experimental.pallas import tpu as pltpu
from jax.experimental.pallas import tpu_sc as plsc

NC = 2
NS = 16
NW = NC * NS


def _sc_gather_build(B, S, P, V, D):
    tokens = B * S
    tpw = tokens // NW
    ck = 56
    sizes = [ck] * (tpw // ck)
    if tpw % ck:
        sizes.append(tpw % ck)
    tail = sizes[-1] if sizes[-1] != ck else None
    wpb = NW // B
    p8 = 8 * ((P + 7) // 8)
    mesh = plsc.VectorSubcoreMesh(core_axis_name="c", subcore_axis_name="s")

    @functools.partial(
        pl.kernel,
        mesh=mesh,
        out_type=jax.ShapeDtypeStruct(((P + S) * B + 8, D), jnp.float32),
        scratch_types=[
            pltpu.VMEM((ck,), jnp.int32),
            pltpu.VMEM((ck,), jnp.int32),
            pltpu.VMEM((ck,), jnp.int32),
            pltpu.VMEM((ck,), jnp.int32),
            pltpu.VMEM((ck, D), jnp.float32),
            pltpu.VMEM((ck, D), jnp.float32),
            pltpu.VMEM((p8,), jnp.int32),
            pltpu.SemaphoreType.DMA,
            pltpu.SemaphoreType.DMA,
            pltpu.SemaphoreType.DMA,
            pltpu.SemaphoreType.DMA,
        ],
        compiler_params=pltpu.CompilerParams(use_tc_tiling_on_sc=True),
    )
    def sc_gather(ids_hbm, table_hbm, sp_hbm, dpos_hbm, pidx_hbm, x_hbm,
                  idx_a, idx_b, didx_a, didx_b, rows_a, rows_b, pidx_v,
                  sem_ga, sem_gb, sem_sa, sem_sb):
        wid = lax.axis_index("s") * NC + lax.axis_index("c")
        idx = [idx_a, idx_b]
        didx = [didx_a, didx_b]
        rows = [rows_a, rows_b]
        sem_g = [sem_ga, sem_gb]
        sem_s = [sem_sa, sem_sb]

        @pl.when(wid % wpb == 0)
        def _():
            batch = wid // wpb
            pltpu.sync_copy(sp_hbm, rows_a.at[pl.ds(0, p8)])
            pltpu.sync_copy(pidx_hbm.at[batch], pidx_v)
            pltpu.async_copy(rows_a.at[pl.ds(0, p8)], x_hbm.at[pidx_v],
                             sem_sa).wait()

        src_base = wid * tpw

        def bufs(k, sz):
            par = k % 2
            if sz == ck:
                return (idx[par], didx[par], rows[par], sem_g[par],
                        sem_s[par])
            return (idx[par].at[pl.ds(0, sz)], didx[par].at[pl.ds(0, sz)],
                    rows[par].at[pl.ds(0, sz)], sem_g[par], sem_s[par])

        offs = []
        o = 0
        for sz in sizes:
            offs.append(o)
            o += sz

        i0, d0, _, _, _ = bufs(0, sizes[0])
        pltpu.sync_copy(ids_hbm.at[pl.ds(src_base, sizes[0])], i0)
        pltpu.sync_copy(dpos_hbm.at[pl.ds(src_base, sizes[0])], d0)

        h_s = [None, None]
        for k, (off, sz) in enumerate(zip(offs, sizes)):
            par = k % 2
            i_r, d_r, r_r, sg, ss = bufs(k, sz)
            h_g = pltpu.async_copy(table_hbm.at[i_r], r_r, sg)
            if k + 1 < len(sizes):
                npar = (k + 1) % 2
                if h_s[npar] is not None:
                    h_s[npar].wait()
                    h_s[npar] = None
                ni, nd, _, _, _ = bufs(k + 1, sizes[k + 1])
                nxt = offs[k + 1]
                pltpu.sync_copy(
                    ids_hbm.at[pl.ds(src_base + nxt, sizes[k + 1])], ni)
                pltpu.sync_copy(
                    dpos_hbm.at[pl.ds(src_base + nxt, sizes[k + 1])], nd)
            h_g.wait()
            h_s[par] = pltpu.async_copy(r_r, x_hbm.at[d_r], ss)
        for h in h_s:
            if h is not None:
                h.wait()

    return sc_gather


def _tc_matmul_build(B, R, D, tile):
    lane = 128
    nt = D // lane
    rt = tile // B
    nj = (R + rt - 1) // rt

    def body(x_ref, w_ref, b_ref, m_ref, o_ref):
        acc = jnp.dot(x_ref[...].astype(jnp.bfloat16),
                      w_ref[...].astype(jnp.bfloat16),
                      preferred_element_type=jnp.float32)
        h = jnp.tanh(acc + b_ref[...])
        mt = m_ref[...].T
        h4 = h.reshape(rt, B, nt, lane) * mt[:, :, None, None]
        o_ref[...] = h4.transpose(0, 2, 1, 3).reshape(rt, nt * B, lane)

    return pl.pallas_call(
        body,
        grid=(nj,),
        in_specs=[
            pl.BlockSpec((tile, D), lambda j: (j, 0)),
            pl.BlockSpec((D, D), lambda j: (0, 0)),
            pl.BlockSpec((1, D), lambda j: (0, 0)),
            pl.BlockSpec((B, rt), lambda j: (0, j)),
        ],
        out_specs=pl.BlockSpec((rt, nt * B, lane), lambda j: (j, 0, 0)),
        out_shape=jax.ShapeDtypeStruct((R, nt * B, lane), jnp.float32),
        compiler_params=pltpu.CompilerParams(
            dimension_semantics=("arbitrary",),
        ),
    )


def kernel(input_ids, attention_mask, token_type_ids, word_embeddings,
           soft_prompt, W, b):
    B, S = input_ids.shape
    V, D = word_embeddings.shape
    P = soft_prompt.shape[0]
    p8 = 8 * ((P + 7) // 8)

    ids = input_ids.reshape(-1).astype(jnp.int32)
    sp_pad = jnp.pad(soft_prompt, ((0, p8 - P), (0, 0)))
    dump = (P + S) * B
    dpos = ((P + jnp.arange(S, dtype=jnp.int32))[None, :] * B
            + jnp.arange(B, dtype=jnp.int32)[:, None]).reshape(-1)
    prow = jnp.arange(p8, dtype=jnp.int32)[None, :]
    pidx = jnp.where(prow < P,
                     prow * B + jnp.arange(B, dtype=jnp.int32)[:, None],
                     dump)

    sc_gather = _sc_gather_build(B, S, P, V, D)
    x = sc_gather(ids, word_embeddings, sp_pad, dpos, pidx)

    mask = jnp.pad(attention_mask.astype(jnp.float32), ((0, 0), (P, 0)),
                   constant_values=1.0)

    tc = _tc_matmul_build(B, P + S, D, 512)
    out3 = tc(x, W, b.reshape(1, D), mask)
    lane = 128
    return (out3.reshape(P + S, D // lane, B, lane)
            .transpose(2, 0, 1, 3).reshape(B, P + S, D))


# speedup vs baseline: 1.0014x; 1.0014x over previous
import functools

import jax
import jax.numpy as jnp
from jax import lax
from jax.experimental import pallas as pl
from jax.experimental.pallas import tpu as pltpu
from jax.experimental.pallas import tpu_sc as plsc

NC = 2
NS = 16
NW = NC * NS


def _sc_gather_build(B, S, P, V, D):
    tokens = B * S
    tpw = tokens // NW
    ck = 56
    sizes = [ck] * (tpw // ck)
    if tpw % ck:
        sizes.append(tpw % ck)
    tail = sizes[-1] if sizes[-1] != ck else None
    wpb = NW // B
    p8 = 8 * ((P + 7) // 8)
    mesh = plsc.VectorSubcoreMesh(core_axis_name="c", subcore_axis_name="s")

    @functools.partial(
        pl.kernel,
        mesh=mesh,
        out_type=jax.ShapeDtypeStruct(((P + S) * B + 8, D), jnp.float32),
        scratch_types=[
            pltpu.VMEM((ck,), jnp.int32),
            pltpu.VMEM((ck,), jnp.int32),
            pltpu.VMEM((ck,), jnp.int32),
            pltpu.VMEM((ck,), jnp.int32),
            pltpu.VMEM((ck, D), jnp.float32),
            pltpu.VMEM((ck, D), jnp.float32),
            pltpu.VMEM((p8,), jnp.int32),
            pltpu.SemaphoreType.DMA,
            pltpu.SemaphoreType.DMA,
            pltpu.SemaphoreType.DMA,
            pltpu.SemaphoreType.DMA,
        ],
        compiler_params=pltpu.CompilerParams(use_tc_tiling_on_sc=True),
    )
    def sc_gather(ids_hbm, table_hbm, sp_hbm, dpos_hbm, pidx_hbm, x_hbm,
                  idx_a, idx_b, didx_a, didx_b, rows_a, rows_b, pidx_v,
                  sem_ga, sem_gb, sem_sa, sem_sb):
        wid = lax.axis_index("s") * NC + lax.axis_index("c")
        idx = [idx_a, idx_b]
        didx = [didx_a, didx_b]
        rows = [rows_a, rows_b]
        sem_g = [sem_ga, sem_gb]
        sem_s = [sem_sa, sem_sb]

        @pl.when(wid % wpb == 0)
        def _():
            batch = wid // wpb
            pltpu.sync_copy(sp_hbm, rows_a.at[pl.ds(0, p8)])
            pltpu.sync_copy(pidx_hbm.at[batch], pidx_v)
            pltpu.async_copy(rows_a.at[pl.ds(0, p8)], x_hbm.at[pidx_v],
                             sem_sa).wait()

        src_base = wid * tpw

        def bufs(k, sz):
            par = k % 2
            if sz == ck:
                return (idx[par], didx[par], rows[par], sem_g[par],
                        sem_s[par])
            return (idx[par].at[pl.ds(0, sz)], didx[par].at[pl.ds(0, sz)],
                    rows[par].at[pl.ds(0, sz)], sem_g[par], sem_s[par])

        offs = []
        o = 0
        for sz in sizes:
            offs.append(o)
            o += sz

        i0, d0, _, _, _ = bufs(0, sizes[0])
        pltpu.sync_copy(ids_hbm.at[pl.ds(src_base, sizes[0])], i0)
        pltpu.sync_copy(dpos_hbm.at[pl.ds(src_base, sizes[0])], d0)

        h_s = [None, None]
        for k, (off, sz) in enumerate(zip(offs, sizes)):
            par = k % 2
            i_r, d_r, r_r, sg, ss = bufs(k, sz)
            h_g = pltpu.async_copy(table_hbm.at[i_r], r_r, sg)
            if k + 1 < len(sizes):
                npar = (k + 1) % 2
                if h_s[npar] is not None:
                    h_s[npar].wait()
                    h_s[npar] = None
                ni, nd, _, _, _ = bufs(k + 1, sizes[k + 1])
                nxt = offs[k + 1]
                pltpu.sync_copy(
                    ids_hbm.at[pl.ds(src_base + nxt, sizes[k + 1])], ni)
                pltpu.sync_copy(
                    dpos_hbm.at[pl.ds(src_base + nxt, sizes[k + 1])], nd)
            h_g.wait()
            h_s[par] = pltpu.async_copy(r_r, x_hbm.at[d_r], ss)
        for h in h_s:
            if h is not None:
                h.wait()

    return sc_gather


def _tc_matmul_build(B, R, D, tile):
    lane = 128
    nt = D // lane
    rt = tile // B
    nj = (R + rt - 1) // rt

    def body(x_ref, w_ref, b_ref, m_ref, o_ref):
        acc = jnp.dot(x_ref[...].astype(jnp.bfloat16),
                      w_ref[...].astype(jnp.bfloat16),
                      preferred_element_type=jnp.float32)
        h = jnp.tanh(acc + b_ref[...]) * m_ref[...]
        o_ref[...] = h.reshape(rt, B, nt, lane).transpose(0, 2, 1, 3).reshape(
            rt, nt * B, lane)

    return pl.pallas_call(
        body,
        grid=(nj,),
        in_specs=[
            pl.BlockSpec((tile, D), lambda j: (j, 0)),
            pl.BlockSpec((D, D), lambda j: (0, 0)),
            pl.BlockSpec((1, D), lambda j: (0, 0)),
            pl.BlockSpec((tile, 1), lambda j: (j, 0)),
        ],
        out_specs=pl.BlockSpec((rt, nt * B, lane), lambda j: (j, 0, 0)),
        out_shape=jax.ShapeDtypeStruct((R, nt * B, lane), jnp.float32),
        compiler_params=pltpu.CompilerParams(
            dimension_semantics=("arbitrary",),
        ),
    )


def kernel(input_ids, attention_mask, token_type_ids, word_embeddings,
           soft_prompt, W, b):
    B, S = input_ids.shape
    V, D = word_embeddings.shape
    P = soft_prompt.shape[0]
    p8 = 8 * ((P + 7) // 8)

    ids = input_ids.reshape(-1).astype(jnp.int32)
    sp_pad = jnp.pad(soft_prompt, ((0, p8 - P), (0, 0)))
    dump = (P + S) * B
    dpos = ((P + jnp.arange(S, dtype=jnp.int32))[None, :] * B
            + jnp.arange(B, dtype=jnp.int32)[:, None]).reshape(-1)
    prow = jnp.arange(p8, dtype=jnp.int32)[None, :]
    pidx = jnp.where(prow < P,
                     prow * B + jnp.arange(B, dtype=jnp.int32)[:, None],
                     dump)

    sc_gather = _sc_gather_build(B, S, P, V, D)
    x = sc_gather(ids, word_embeddings, sp_pad, dpos, pidx)

    mask = jnp.concatenate(
        [jnp.ones((B, P), dtype=attention_mask.dtype), attention_mask], axis=1
    ).astype(jnp.float32).T.reshape((P + S) * B, 1)

    tc = _tc_matmul_build(B, P + S, D, 512)
    out3 = tc(x, W, b.reshape(1, D), mask)
    lane = 128
    return (out3.reshape(P + S, D // lane, B, lane)
            .transpose(2, 0, 1, 3).reshape(B, P + S, D))
